# Initial kernel scaffold; baseline (speedup 1.0000x reference)
import jax
import jax.numpy as jnp
from jax.experimental import pallas as pl


def kernel(node_features, edge_features, edge_index, node_table, edge_table, w_W, w_b, edgew_W, edgew_b, attn_W, attn_b, ln_gamma, ln_beta):
    raise NotImplementedError("write your pallas kernel here")


# speedup vs baseline: 4.1463x; 4.1463x over previous
import functools

import jax
import jax.numpy as jnp
from jax import lax
from jax.experimental import pallas as pl
from jax.experimental.pallas import tpu as pltpu
from jax.experimental.pallas import tpu_sc as plsc

N = 10000
E = 320000
D = 128
EDICT = 1000
SLOPE = 0.2
LN_EPS = 1e-5

NC, NS = 2, 16
NW = NC * NS
PT = E // NW
C = 80
NCH = PT // C
NPAD = 10240
GPT = NPAD // NW

_MESH = dict(core_axis_name="c", subcore_axis_name="s", num_cores=NC,
             num_subcores=NS)


def _nf_gather_body(idx_hbm, table_hbm, out_hbm, idxv, rows, sem):
    wid = lax.axis_index("s") * NC + lax.axis_index("c")

    def body(i, _):
        base = wid * GPT + i * C
        pltpu.sync_copy(idx_hbm.at[pl.ds(base, C)], idxv)
        pltpu.async_copy(table_hbm.at[idxv], rows, sem).wait()
        pltpu.sync_copy(rows, out_hbm.at[pl.ds(base, C)])
        return 0

    lax.fori_loop(0, GPT // C, body, 0)


def _nf_gather(idx_pad, node_table):
    mesh = plsc.VectorSubcoreMesh(**_MESH)
    f = pl.kernel(
        _nf_gather_body,
        out_type=jax.ShapeDtypeStruct((NPAD, D), jnp.float32),
        mesh=mesh,
        scratch_types=[
            pltpu.VMEM((C,), jnp.int32),
            pltpu.VMEM((C, D), jnp.float32),
            pltpu.SemaphoreType.DMA,
        ],
    )
    return f(idx_pad, node_table)


def _m1_body(nf_ref, wt_ref, wbo_ref, wb_ref, et_ref, ewW_ref, ewb_ref,
             ae_ref, ab_ref, g_ref, be_ref,
             A_ref, B_ref, eatt_ref, teft_ref):
    nf = nf_ref[...]
    A_ref[...] = (jnp.dot(nf, wt_ref[...], preferred_element_type=jnp.float32)
                  + wb_ref[...])
    B_ref[...] = jnp.dot(nf, wbo_ref[...], preferred_element_type=jnp.float32)
    eh = (jnp.dot(et_ref[...], ewW_ref[...],
                  preferred_element_type=jnp.float32) + ewb_ref[...])
    lr = jnp.maximum(eh, eh * SLOPE)
    eatt_ref[...] = (jnp.dot(lr, ae_ref[...],
                             preferred_element_type=jnp.float32) + ab_ref[...])
    mu = jnp.mean(eh, axis=-1, keepdims=True)
    var = jnp.mean((eh - mu) ** 2, axis=-1, keepdims=True)
    teft_ref[...] = (eh - mu) * lax.rsqrt(var + LN_EPS) * g_ref[...] + be_ref[...]


def _m1(nf, w_W, w_b, edge_table, edgew_W, edgew_b, attn_W, attn_b,
        ln_gamma, ln_beta):
    out_shape = [
        jax.ShapeDtypeStruct((N, D), jnp.float32),
        jax.ShapeDtypeStruct((N, D), jnp.float32),
        jax.ShapeDtypeStruct((EDICT, 1), jnp.float32),
        jax.ShapeDtypeStruct((EDICT, D), jnp.float32),
    ]
    return pl.pallas_call(_m1_body, out_shape=out_shape)(
        nf, w_W[:D], w_W[D:], w_b.reshape(1, D),
        edge_table, edgew_W, edgew_b.reshape(1, D),
        attn_W[D:].reshape(D, 1), attn_b.reshape(1, 1),
        ln_gamma.reshape(1, D), ln_beta.reshape(1, D))


def _edge_body(src_hbm, dst_hbm, ef_hbm, A_hbm, B_hbm, NF_hbm, ah_hbm,
               eatt_hbm, teft_hbm,
               out0a, out0b, s0, s1, deg0, deg1, tef_hbm,
               idxs, idxd, idxe, bufA, bufB, bufNF, bufT, wbuf, onesv,
               zv, ahv, eattv, out0_sh, s_sh, deg_sh,
               semA, semB, semN, semT):
    core = lax.axis_index("c")
    sid = lax.axis_index("s")
    wid = sid * NC + core

    pltpu.sync_copy(ah_hbm, ahv)
    pltpu.sync_copy(eatt_hbm, eattv)

    zero16 = jnp.zeros((16,), jnp.float32)
    one16 = jnp.ones((16,), jnp.float32)
    for i in range(C // 16):
        zv[pl.ds(i * 16, 16)] = zero16
        onesv[pl.ds(i * 16, 16)] = one16

    def zrow(r, _):
        for dd in range(D // 16):
            bufNF[r, pl.ds(dd * 16, 16)] = zero16
        return 0

    lax.fori_loop(0, C, zrow, 0)

    @pl.when(sid == 0)
    def _():
        def zs(i, _):
            pltpu.sync_copy(zv, s_sh.at[pl.ds(i * C, C)])
            return 0
        lax.fori_loop(0, N // C, zs, 0)

    @pl.when(sid == 1)
    def _():
        def zd(i, _):
            pltpu.sync_copy(zv, deg_sh.at[pl.ds(i * C, C)])
            return 0
        lax.fori_loop(0, N // C, zd, 0)

    zbase = pl.multiple_of(sid * 624, 8)
    for j in range(8):
        pltpu.sync_copy(bufNF, out0_sh.at[pl.ds(zbase + j * C, C)])

    plsc.subcore_barrier()

    def chunk(ci, _):
        ebase = wid * PT + ci * C
        pltpu.sync_copy(src_hbm.at[pl.ds(ebase, C)], idxs)
        pltpu.sync_copy(dst_hbm.at[pl.ds(ebase, C)], idxd)
        pltpu.sync_copy(ef_hbm.at[pl.ds(ebase, C)], idxe)
        cpA = pltpu.async_copy(A_hbm.at[idxd], bufA, semA)
        cpB = pltpu.async_copy(B_hbm.at[idxs], bufB, semB)
        cpN = pltpu.async_copy(NF_hbm.at[idxs], bufNF, semN)
        cpT = pltpu.async_copy(teft_hbm.at[idxe], bufT, semT)
        cpA.wait()
        cpB.wait()

        iota16 = lax.iota(jnp.int32, 16)

        def group(g, _):
            rows = iota16 + g * 16
            acc = zero16
            for dd in range(D // 16):
                ahblk = ahv[pl.ds(dd * 16, 16)]
                for i in range(16):
                    cols = jnp.full((16,), dd * 16 + i, jnp.int32)
                    va = plsc.load_gather(bufA, [rows, cols])
                    vb = plsc.load_gather(bufB, [rows, cols])
                    h = va + vb
                    acc = acc + jnp.maximum(h, h * SLOPE) * ahblk[i]
            e16 = idxe[pl.ds(g * 16, 16)]
            ea = plsc.load_gather(eattv, [e16])
            w16 = jnp.exp(acc + ea)
            wbuf[pl.ds(g * 16, 16)] = w16
            return 0

        lax.fori_loop(0, C // 16, group, 0)

        pltpu.sync_copy(wbuf, s_sh.at[idxd], add=True)
        pltpu.sync_copy(onesv, deg_sh.at[idxd], add=True)

        cpN.wait()

        def sgroup(g, _):
            w16 = wbuf[pl.ds(g * 16, 16)]
            for l in range(16):
                e = g * 16 + l
                w = w16[l]
                for dd in range(D // 16):
                    sl = pl.ds(dd * 16, 16)
                    bufNF[e, sl] = bufNF[e, sl] * w
            return 0

        lax.fori_loop(0, C // 16, sgroup, 0)
        pltpu.sync_copy(bufNF, out0_sh.at[idxd], add=True)

        cpT.wait()
        pltpu.sync_copy(bufT, tef_hbm.at[pl.ds(ebase, C)])
        return 0

    lax.fori_loop(0, NCH, chunk, 0)

    plsc.subcore_barrier()

    obase = pl.multiple_of(sid * 624, 8)

    @pl.when(core == 0)
    def _():
        pltpu.sync_copy(out0_sh.at[pl.ds(obase, 640)],
                        out0a.at[pl.ds(obase, 640)])

    @pl.when(core == 1)
    def _():
        pltpu.sync_copy(out0_sh.at[pl.ds(obase, 640)],
                        out0b.at[pl.ds(obase, 640)])

    @pl.when(jnp.logical_and(sid == 10, core == 0))
    def _():
        pltpu.sync_copy(s_sh, s0)

    @pl.when(jnp.logical_and(sid == 10, core == 1))
    def _():
        pltpu.sync_copy(s_sh, s1)

    @pl.when(jnp.logical_and(sid == 11, core == 0))
    def _():
        pltpu.sync_copy(deg_sh, deg0)

    @pl.when(jnp.logical_and(sid == 11, core == 1))
    def _():
        pltpu.sync_copy(deg_sh, deg1)


def _edge_kernel(src, dst, ef, A, B, nf, ah, eatt, teft):
    mesh = plsc.VectorSubcoreMesh(**_MESH)
    f = pl.kernel(
        _edge_body,
        out_type=[
            jax.ShapeDtypeStruct((N, D), jnp.float32),
            jax.ShapeDtypeStruct((N, D), jnp.float32),
            jax.ShapeDtypeStruct((N,), jnp.float32),
            jax.ShapeDtypeStruct((N,), jnp.float32),
            jax.ShapeDtypeStruct((N,), jnp.float32),
            jax.ShapeDtypeStruct((N,), jnp.float32),
            jax.ShapeDtypeStruct((E, D), jnp.float32),
        ],
        mesh=mesh,
        scratch_types=[
            pltpu.VMEM((C,), jnp.int32),
            pltpu.VMEM((C,), jnp.int32),
            pltpu.VMEM((C,), jnp.int32),
            pltpu.VMEM((C, D), jnp.float32),
            pltpu.VMEM((C, D), jnp.float32),
            pltpu.VMEM((C, D), jnp.float32),
            pltpu.VMEM((C, D), jnp.float32),
            pltpu.VMEM((C,), jnp.float32),
            pltpu.VMEM((C,), jnp.float32),
            pltpu.VMEM((C,), jnp.float32),
            pltpu.VMEM((D,), jnp.float32),
            pltpu.VMEM((EDICT,), jnp.float32),
            pltpu.VMEM_SHARED((N, D), jnp.float32),
            pltpu.VMEM_SHARED((N,), jnp.float32),
            pltpu.VMEM_SHARED((N,), jnp.float32),
            pltpu.SemaphoreType.DMA,
            pltpu.SemaphoreType.DMA,
            pltpu.SemaphoreType.DMA,
            pltpu.SemaphoreType.DMA,
        ],
        compiler_params=pltpu.CompilerParams(needs_layout_passes=False),
    )
    return f(src, dst, ef, A, B, nf, ah, eatt, teft)


def _fin_body(o0a_ref, o0b_ref, s0_ref, s1_ref, d0_ref, d1_ref, nf_ref,
              g_ref, be_ref, out_ref):
    o = o0a_ref[...] + o0b_ref[...]
    s = s0_ref[...] + s1_ref[...]
    dg = d0_ref[...] + d1_ref[...]
    pre = o / jnp.where(s > 0.0, s, 1.0) + dg * nf_ref[...]
    mu = jnp.mean(pre, axis=-1, keepdims=True)
    var = jnp.mean((pre - mu) ** 2, axis=-1, keepdims=True)
    out_ref[...] = (pre - mu) * lax.rsqrt(var + LN_EPS) * g_ref[...] + be_ref[...]


def _fin(out0a, out0b, s0, s1, deg0, deg1, nf, ln_gamma, ln_beta):
    return pl.pallas_call(
        _fin_body,
        out_shape=jax.ShapeDtypeStruct((N, D), jnp.float32),
    )(out0a, out0b, s0.reshape(N, 1), s1.reshape(N, 1),
      deg0.reshape(N, 1), deg1.reshape(N, 1), nf,
      ln_gamma.reshape(1, D), ln_beta.reshape(1, D))


def kernel(node_features, edge_features, edge_index, node_table, edge_table,
           w_W, w_b, edgew_W, edgew_b, attn_W, attn_b, ln_gamma, ln_beta):
    nfeat = node_features.astype(jnp.int32)
    ef = edge_features.astype(jnp.int32)
    src = edge_index[0].astype(jnp.int32)
    dst = edge_index[1].astype(jnp.int32)

    idx_pad = jnp.pad(nfeat, (0, NPAD - N))
    nf = _nf_gather(idx_pad, node_table)[:N]

    A, B, eatt2, teft = _m1(nf, w_W, w_b, edge_table, edgew_W, edgew_b,
                            attn_W, attn_b, ln_gamma, ln_beta)
    eatt = eatt2.reshape(EDICT)
    ah = attn_W[:D, 0]

    out0a, out0b, s0, s1, deg0, deg1, tef = _edge_kernel(
        src, dst, ef, A, B, nf, ah, eatt, teft)
    out = _fin(out0a, out0b, s0, s1, deg0, deg1, nf, ln_gamma, ln_beta)
    return (out, tef)


# speedup vs baseline: 9.5363x; 2.2999x over previous
import functools

import jax
import jax.numpy as jnp
from jax import lax
from jax.experimental import pallas as pl
from jax.experimental.pallas import tpu as pltpu
from jax.experimental.pallas import tpu_sc as plsc

N = 10000
E = 320000
D = 128
EDICT = 1000
SLOPE = 0.2
LN_EPS = 1e-5

NC, NS = 2, 16
NW = NC * NS
PT = E // NW
C = 80
NCH = PT // C
NPAD = 10240
GPT = NPAD // NW

_MESH = dict(core_axis_name="c", subcore_axis_name="s", num_cores=NC,
             num_subcores=NS)


def _nf_gather_body(idx_hbm, table_hbm, out_hbm, idxv, rows, sem):
    wid = lax.axis_index("s") * NC + lax.axis_index("c")

    def body(i, _):
        base = wid * GPT + i * C
        pltpu.sync_copy(idx_hbm.at[pl.ds(base, C)], idxv)
        pltpu.async_copy(table_hbm.at[idxv], rows, sem).wait()
        pltpu.sync_copy(rows, out_hbm.at[pl.ds(base, C)])
        return 0

    lax.fori_loop(0, GPT // C, body, 0)


def _nf_gather(idx_pad, node_table):
    mesh = plsc.VectorSubcoreMesh(**_MESH)
    f = pl.kernel(
        _nf_gather_body,
        out_type=jax.ShapeDtypeStruct((NPAD, D), jnp.float32),
        mesh=mesh,
        scratch_types=[
            pltpu.VMEM((C,), jnp.int32),
            pltpu.VMEM((C, D), jnp.float32),
            pltpu.SemaphoreType.DMA,
        ],
    )
    return f(idx_pad, node_table)


def _m1_body(nf_ref, wt_ref, wbo_ref, wb_ref, et_ref, ewW_ref, ewb_ref,
             ae_ref, ab_ref, g_ref, be_ref,
             A_ref, B_ref, eatt_ref, teft_ref):
    nf = nf_ref[...]
    A_ref[...] = (jnp.dot(nf, wt_ref[...], preferred_element_type=jnp.float32)
                  + wb_ref[...])
    B_ref[...] = jnp.dot(nf, wbo_ref[...], preferred_element_type=jnp.float32)
    eh = (jnp.dot(et_ref[...], ewW_ref[...],
                  preferred_element_type=jnp.float32) + ewb_ref[...])
    lr = jnp.maximum(eh, eh * SLOPE)
    eatt_ref[...] = (jnp.dot(lr, ae_ref[...],
                             preferred_element_type=jnp.float32) + ab_ref[...])
    mu = jnp.mean(eh, axis=-1, keepdims=True)
    var = jnp.mean((eh - mu) ** 2, axis=-1, keepdims=True)
    teft_ref[...] = (eh - mu) * lax.rsqrt(var + LN_EPS) * g_ref[...] + be_ref[...]


def _m1(nf, w_W, w_b, edge_table, edgew_W, edgew_b, attn_W, attn_b,
        ln_gamma, ln_beta):
    out_shape = [
        jax.ShapeDtypeStruct((N, D), jnp.float32),
        jax.ShapeDtypeStruct((N, D), jnp.float32),
        jax.ShapeDtypeStruct((EDICT, 1), jnp.float32),
        jax.ShapeDtypeStruct((EDICT, D), jnp.float32),
    ]
    return pl.pallas_call(_m1_body, out_shape=out_shape)(
        nf, w_W[:D], w_W[D:], w_b.reshape(1, D),
        edge_table, edgew_W, edgew_b.reshape(1, D),
        attn_W[D:].reshape(D, 1), attn_b.reshape(1, 1),
        ln_gamma.reshape(1, D), ln_beta.reshape(1, D))


def _sc_zero_vec(zv, val16):
    for i in range(C // 16):
        zv[pl.ds(i * 16, 16)] = val16


def _attn_body(src_hbm, dst_hbm, ef_hbm, A_hbm, B_hbm, ah_hbm, eatt_hbm,
               teft_hbm,
               s0, s1, deg0, deg1, w_hbm, tef_hbm,
               idxs0, idxd0, idxe0, idxs1, idxd1, idxe1,
               bufA0, bufB0, bufT0, bufA1, bufB1, bufT1,
               wbuf0, wbuf1,
               onesv, zv, ahv, eattv, scr, s_sh, deg_sh,
               semI0, semI1, semG0, semG1, semW0, semW1):
    core = lax.axis_index("c")
    sid = lax.axis_index("s")
    wid = sid * NC + core

    IDX = [(idxs0, idxd0, idxe0), (idxs1, idxd1, idxe1)]
    BUF = [(bufA0, bufB0, bufT0), (bufA1, bufB1, bufT1)]
    WB = [wbuf0, wbuf1]
    SEMI = [semI0, semI1]
    SEMG = [semG0, semG1]
    SEMW = [semW0, semW1]

    pltpu.sync_copy(ah_hbm, ahv)
    pltpu.sync_copy(eatt_hbm, eattv)

    zero16 = jnp.zeros((16,), jnp.float32)
    one16 = jnp.ones((16,), jnp.float32)
    _sc_zero_vec(zv, zero16)
    _sc_zero_vec(onesv, one16)

    @pl.when(sid == 0)
    def _():
        def zs(i, _):
            pltpu.sync_copy(zv, s_sh.at[pl.ds(i * C, C)])
            return 0
        lax.fori_loop(0, N // C, zs, 0)

    @pl.when(sid == 1)
    def _():
        def zd(i, _):
            pltpu.sync_copy(zv, deg_sh.at[pl.ds(i * C, C)])
            return 0
        lax.fori_loop(0, N // C, zd, 0)

    plsc.subcore_barrier()

    iota16 = lax.iota(jnp.int32, 16)

    def issue_idx(c, sl):
        ebase = wid * PT + c * C
        si, di, ei = IDX[sl]
        return [pltpu.async_copy(src_hbm.at[pl.ds(ebase, C)], si, SEMI[sl]),
                pltpu.async_copy(dst_hbm.at[pl.ds(ebase, C)], di, SEMI[sl]),
                pltpu.async_copy(ef_hbm.at[pl.ds(ebase, C)], ei, SEMI[sl])]

    def issue_gathers(sl):
        si, di, ei = IDX[sl]
        bA, bB, bT = BUF[sl]
        return [pltpu.async_copy(A_hbm.at[di], bA, SEMG[sl]),
                pltpu.async_copy(B_hbm.at[si], bB, SEMG[sl]),
                pltpu.async_copy(teft_hbm.at[ei], bT, SEMG[sl])]

    def compute(sl):
        _, _, ei = IDX[sl]
        bA, bB, _ = BUF[sl]
        wb = WB[sl]

        ah_regs = [ahv[pl.ds(dd * 16, 16)] for dd in range(D // 16)]

        def group(g, _):
            for l in range(16):
                e = g * 16 + l
                acc = zero16
                for dd in range(D // 16):
                    s_ = pl.ds(dd * 16, 16)
                    h = bA[e, s_] + bB[e, s_]
                    acc = acc + jnp.maximum(h, h * SLOPE) * ah_regs[dd]
                scr[l, pl.ds(0, 16)] = acc
            tot = zero16
            for cc in range(16):
                col = plsc.load_gather(scr, [iota16,
                                             jnp.full((16,), cc, jnp.int32)])
                tot = tot + col
            ea = plsc.load_gather(eattv, [ei[pl.ds(g * 16, 16)]])
            w16 = jnp.exp(tot + ea)
            wb[pl.ds(g * 16, 16)] = w16
            return 0

        lax.fori_loop(0, C // 16, group, 0)

    def halfchunk(c, sl, g):
        _, di, _ = IDX[sl]
        _, _, bT = BUF[sl]
        ebase = wid * PT + c * C
        g[0].wait()
        g[1].wait()
        compute(sl)
        pltpu.sync_copy(WB[sl], s_sh.at[di], add=True)
        pltpu.sync_copy(onesv, deg_sh.at[di], add=True)
        g[2].wait()
        return [pltpu.async_copy(bT, tef_hbm.at[pl.ds(ebase, C)], SEMW[sl]),
                pltpu.async_copy(WB[sl], w_hbm.at[pl.ds(ebase, C)], SEMW[sl])]

    def pair(k, _):
        c0 = 2 * k
        c1 = 2 * k + 1
        i0 = issue_idx(c0, 0)
        i1 = issue_idx(c1, 1)
        for cp in i0:
            cp.wait()
        g0 = issue_gathers(0)
        for cp in i1:
            cp.wait()
        g1 = issue_gathers(1)
        w0 = halfchunk(c0, 0, g0)
        w1 = halfchunk(c1, 1, g1)
        for cp in w0:
            cp.wait()
        for cp in w1:
            cp.wait()
        return 0

    lax.fori_loop(0, NCH // 2, pair, 0)

    ct = NCH - 1
    it = issue_idx(ct, 0)
    for cp in it:
        cp.wait()
    gt = issue_gathers(0)
    wt = halfchunk(ct, 0, gt)
    for cp in wt:
        cp.wait()

    plsc.subcore_barrier()

    @pl.when(jnp.logical_and(sid == 10, core == 0))
    def _():
        pltpu.sync_copy(s_sh, s0)

    @pl.when(jnp.logical_and(sid == 10, core == 1))
    def _():
        pltpu.sync_copy(s_sh, s1)

    @pl.when(jnp.logical_and(sid == 11, core == 0))
    def _():
        pltpu.sync_copy(deg_sh, deg0)

    @pl.when(jnp.logical_and(sid == 11, core == 1))
    def _():
        pltpu.sync_copy(deg_sh, deg1)


def _attn_kernel(src, dst, ef, A, B, ah, eatt, teft):
    mesh = plsc.VectorSubcoreMesh(**_MESH)
    idx_t = pltpu.VMEM((C,), jnp.int32)
    buf_t = pltpu.VMEM((C, D), jnp.float32)
    w_t = pltpu.VMEM((C,), jnp.float32)
    f = pl.kernel(
        _attn_body,
        out_type=[
            jax.ShapeDtypeStruct((N,), jnp.float32),
            jax.ShapeDtypeStruct((N,), jnp.float32),
            jax.ShapeDtypeStruct((N,), jnp.float32),
            jax.ShapeDtypeStruct((N,), jnp.float32),
            jax.ShapeDtypeStruct((E,), jnp.float32),
            jax.ShapeDtypeStruct((E, D), jnp.float32),
        ],
        mesh=mesh,
        scratch_types=(
            [idx_t] * 6
            + [buf_t] * 6
            + [w_t] * 2
            + [
                pltpu.VMEM((C,), jnp.float32),
                pltpu.VMEM((C,), jnp.float32),
                pltpu.VMEM((D,), jnp.float32),
                pltpu.VMEM((EDICT,), jnp.float32),
                pltpu.VMEM((16, 16), jnp.float32),
                pltpu.VMEM_SHARED((N,), jnp.float32),
                pltpu.VMEM_SHARED((N,), jnp.float32),
            ]
            + [pltpu.SemaphoreType.DMA] * 6
        ),
        compiler_params=pltpu.CompilerParams(needs_layout_passes=False),
    )
    return f(src, dst, ef, A, B, ah, eatt, teft)


def _msg_body(src_hbm, dst_hbm, NF_hbm, w_hbm,
              out0a, out0b,
              idxs0, idxd0, idxs1, idxd1,
              bufN0, bufN1, wbuf0, wbuf1,
              out0_sh,
              semI0, semI1, semG0, semG1):
    core = lax.axis_index("c")
    sid = lax.axis_index("s")
    wid = sid * NC + core

    IDX = [(idxs0, idxd0), (idxs1, idxd1)]
    BUF = [bufN0, bufN1]
    WB = [wbuf0, wbuf1]
    SEMI = [semI0, semI1]
    SEMG = [semG0, semG1]

    zero16 = jnp.zeros((16,), jnp.float32)

    def zrow(r, _):
        for dd in range(D // 16):
            bufN0[r, pl.ds(dd * 16, 16)] = zero16
        return 0

    lax.fori_loop(0, C, zrow, 0)

    zbase = pl.multiple_of(sid * 624, 8)
    for j in range(8):
        pltpu.sync_copy(bufN0, out0_sh.at[pl.ds(zbase + j * C, C)])

    plsc.subcore_barrier()

    def issue_idx(c, sl):
        ebase = wid * PT + c * C
        si, di = IDX[sl]
        return [pltpu.async_copy(src_hbm.at[pl.ds(ebase, C)], si, SEMI[sl]),
                pltpu.async_copy(dst_hbm.at[pl.ds(ebase, C)], di, SEMI[sl])]

    def issue_gathers(c, sl):
        si, _ = IDX[sl]
        ebase = wid * PT + c * C
        return [pltpu.async_copy(NF_hbm.at[si], BUF[sl], SEMG[sl]),
                pltpu.async_copy(w_hbm.at[pl.ds(ebase, C)], WB[sl], SEMG[sl])]

    def compute(sl):
        bN = BUF[sl]
        wb = WB[sl]

        def sgroup(g, _):
            w16 = wb[pl.ds(g * 16, 16)]
            for l in range(16):
                e = g * 16 + l
                w = w16[l]
                for dd in range(D // 16):
                    s_ = pl.ds(dd * 16, 16)
                    bN[e, s_] = bN[e, s_] * w
            return 0

        lax.fori_loop(0, C // 16, sgroup, 0)

    def halfchunk(sl, g):
        _, di = IDX[sl]
        for cp in g:
            cp.wait()
        compute(sl)
        pltpu.sync_copy(BUF[sl], out0_sh.at[di], add=True)

    def pair(k, _):
        c0 = 2 * k
        c1 = 2 * k + 1
        i0 = issue_idx(c0, 0)
        i1 = issue_idx(c1, 1)
        for cp in i0:
            cp.wait()
        g0 = issue_gathers(c0, 0)
        for cp in i1:
            cp.wait()
        g1 = issue_gathers(c1, 1)
        halfchunk(0, g0)
        halfchunk(1, g1)
        return 0

    lax.fori_loop(0, NCH // 2, pair, 0)

    ct = NCH - 1
    it = issue_idx(ct, 0)
    for cp in it:
        cp.wait()
    gt = issue_gathers(ct, 0)
    halfchunk(0, gt)

    plsc.subcore_barrier()

    obase = pl.multiple_of(sid * 624, 8)

    @pl.when(core == 0)
    def _():
        pltpu.sync_copy(out0_sh.at[pl.ds(obase, 640)],
                        out0a.at[pl.ds(obase, 640)])

    @pl.when(core == 1)
    def _():
        pltpu.sync_copy(out0_sh.at[pl.ds(obase, 640)],
                        out0b.at[pl.ds(obase, 640)])


def _msg_kernel(src, dst, nf, w_e):
    mesh = plsc.VectorSubcoreMesh(**_MESH)
    idx_t = pltpu.VMEM((C,), jnp.int32)
    buf_t = pltpu.VMEM((C, D), jnp.float32)
    w_t = pltpu.VMEM((C,), jnp.float32)
    f = pl.kernel(
        _msg_body,
        out_type=[
            jax.ShapeDtypeStruct((N, D), jnp.float32),
            jax.ShapeDtypeStruct((N, D), jnp.float32),
        ],
        mesh=mesh,
        scratch_types=(
            [idx_t] * 4
            + [buf_t] * 2
            + [w_t] * 2
            + [pltpu.VMEM_SHARED((N, D), jnp.float32)]
            + [pltpu.SemaphoreType.DMA] * 4
        ),
        compiler_params=pltpu.CompilerParams(needs_layout_passes=False),
    )
    return f(src, dst, nf, w_e)


def _fin_body(o0a_ref, o0b_ref, s0_ref, s1_ref, d0_ref, d1_ref, nf_ref,
              g_ref, be_ref, out_ref):
    o = o0a_ref[...] + o0b_ref[...]
    s = s0_ref[...] + s1_ref[...]
    dg = d0_ref[...] + d1_ref[...]
    pre = o / jnp.where(s > 0.0, s, 1.0) + dg * nf_ref[...]
    mu = jnp.mean(pre, axis=-1, keepdims=True)
    var = jnp.mean((pre - mu) ** 2, axis=-1, keepdims=True)
    out_ref[...] = (pre - mu) * lax.rsqrt(var + LN_EPS) * g_ref[...] + be_ref[...]


def _fin(out0a, out0b, s0, s1, deg0, deg1, nf, ln_gamma, ln_beta):
    return pl.pallas_call(
        _fin_body,
        out_shape=jax.ShapeDtypeStruct((N, D), jnp.float32),
    )(out0a, out0b, s0.reshape(N, 1), s1.reshape(N, 1),
      deg0.reshape(N, 1), deg1.reshape(N, 1), nf,
      ln_gamma.reshape(1, D), ln_beta.reshape(1, D))


def kernel(node_features, edge_features, edge_index, node_table, edge_table,
           w_W, w_b, edgew_W, edgew_b, attn_W, attn_b, ln_gamma, ln_beta):
    nfeat = node_features.astype(jnp.int32)
    ef = edge_features.astype(jnp.int32)
    src = edge_index[0].astype(jnp.int32)
    dst = edge_index[1].astype(jnp.int32)

    idx_pad = jnp.pad(nfeat, (0, NPAD - N))
    nf = _nf_gather(idx_pad, node_table)[:N]

    A, B, eatt2, teft = _m1(nf, w_W, w_b, edge_table, edgew_W, edgew_b,
                            attn_W, attn_b, ln_gamma, ln_beta)
    eatt = eatt2.reshape(EDICT)
    ah = attn_W[:D, 0]

    s0, s1, deg0, deg1, w_e, tef = _attn_kernel(
        src, dst, ef, A, B, ah, eatt, teft)
    out0a, out0b = _msg_kernel(src, dst, nf, w_e)
    out = _fin(out0a, out0b, s0, s1, deg0, deg1, nf, ln_gamma, ln_beta)
    return (out, tef)
